# Initial kernel scaffold; baseline (speedup 1.0000x reference)
import jax
import jax.numpy as jnp
from jax.experimental import pallas as pl


def kernel(x, W0, b0, W1, as1, ad1, bv1, W2, as2, ad2, bv2, W3, as3, ad3, bv3, g1, be1, g2, be2, g3, be3, Wc1, bc1, Wc2, bc2, edge_index):
    raise NotImplementedError("write your pallas kernel here")


# speedup vs baseline: 15.6377x; 15.6377x over previous
import functools

import jax
import jax.numpy as jnp
from jax import lax
from jax.experimental import pallas as pl
from jax.experimental.pallas import tpu as pltpu
from jax.experimental.pallas import tpu_sc as plsc

N = 50000
NPAD = 50176
HALF = 25088
TROWS = HALF + 16
E = 800000
EPAD = 802816
PER_TILE_E = EPAD // 16
NCHUNK = PER_TILE_E // 128
BLK = 3136
GRID = NPAD // BLK
F32 = jnp.float32
I32 = jnp.int32


def _dense0_body(xr, wr, br, outr):
    y = jnp.dot(xr[...], wr[...], preferred_element_type=F32) + br[...]
    outr[...] = jnp.maximum(y, 0.0)


def _make_pre_body(H):
    C = 64 // H

    def _pre_body(hr, wr, asr, adr, xpr, as8r, ad8r, gmr):
        xp = jnp.dot(hr[...], wr[...], preferred_element_type=F32)
        xpr[...] = xp
        pa = xp * asr[...]
        pd = xp * adr[...]
        a8 = jnp.zeros((BLK, 8), F32)
        d8 = jnp.zeros((BLK, 8), F32)
        for h in range(H):
            oh = (lax.broadcasted_iota(I32, (1, 8), 1) == h).astype(F32)
            a8 = a8 + jnp.sum(pa[:, h * C:(h + 1) * C], axis=1,
                              keepdims=True) * oh
            d8 = d8 + jnp.sum(pd[:, h * C:(h + 1) * C], axis=1,
                              keepdims=True) * oh
        as8r[...] = a8
        ad8r[...] = d8
        bm = jnp.max(a8, axis=0, keepdims=True)

        @pl.when(pl.program_id(0) == 0)
        def _():
            gmr[...] = bm

        @pl.when(pl.program_id(0) != 0)
        def _():
            gmr[...] = jnp.maximum(gmr[...], bm)

    return _pre_body


def _mid_body(adr, asr, gmr, mr):
    t = gmr[...] + adr[...]
    bound = jnp.maximum(t, 0.2 * t)
    ts = asr[...] + adr[...]
    aself = jnp.maximum(ts, 0.2 * ts)
    mr[...] = jnp.maximum(aself, bound - 60.0)


def _head_body(hr, w1r, b1r, w2r, b2r, outr):
    y = jnp.dot(hr[...], w1r[...], preferred_element_type=F32) + b1r[...]
    y = jnp.maximum(y, 0.0)
    outr[...] = jnp.dot(y, w2r[...], preferred_element_type=F32) + b2r[...]


_rowspec = lambda w: pl.BlockSpec((BLK, w), lambda r: (r, 0))
_fullspec = lambda a, b: pl.BlockSpec((a, b), lambda r: (0, 0))

_dense0 = pl.pallas_call(
    _dense0_body,
    grid=(GRID,),
    in_specs=[_rowspec(8), _fullspec(8, 64), _fullspec(1, 64)],
    out_specs=_rowspec(64),
    out_shape=jax.ShapeDtypeStruct((NPAD, 64), F32),
)

_pre = {H: pl.pallas_call(
    _make_pre_body(H),
    grid=(GRID,),
    in_specs=[_rowspec(64), _fullspec(64, 64), _fullspec(1, 64),
              _fullspec(1, 64)],
    out_specs=[_rowspec(64), _rowspec(8), _rowspec(8), _fullspec(1, 8)],
    out_shape=[jax.ShapeDtypeStruct((NPAD, 64), F32),
               jax.ShapeDtypeStruct((NPAD, 8), F32),
               jax.ShapeDtypeStruct((NPAD, 8), F32),
               jax.ShapeDtypeStruct((1, 8), F32)],
) for H in (4, 1)}

_mid = pl.pallas_call(
    _mid_body,
    grid=(GRID,),
    in_specs=[_rowspec(8), _rowspec(8), _fullspec(1, 8)],
    out_specs=_rowspec(8),
    out_shape=jax.ShapeDtypeStruct((NPAD, 8), F32),
)

_head = pl.pallas_call(
    _head_body,
    grid=(GRID,),
    in_specs=[_rowspec(64), _fullspec(64, 32), _fullspec(1, 32),
              _fullspec(32, 8), _fullspec(1, 8)],
    out_specs=_rowspec(8),
    out_shape=jax.ShapeDtypeStruct((NPAD, 8), F32),
)


def _sc_exp(x):
    t = x * jnp.float32(1.4426950408889634)
    i = (t + jnp.float32(512.5)).astype(I32) - 512
    f = t - i.astype(F32)
    r = f * jnp.float32(0.6931471805599453)
    p = jnp.full_like(r, 1.0 / 40320)
    for cinv in (1.0 / 5040, 1.0 / 720, 1.0 / 120, 1.0 / 24, 1.0 / 6, 0.5,
                 1.0, 1.0):
        p = p * r + jnp.float32(cinv)
    ic = jnp.clip(i, -126, 127)
    scale = jax.lax.bitcast_convert_type(
        jax.lax.shift_left(ic + 127, 23), F32)
    return p * scale


def _lane_gather(v, idx):
    return lax.gather(
        v, idx[:, None],
        lax.GatherDimensionNumbers(offset_dims=(),
                                   collapsed_slice_dims=(0,),
                                   start_index_map=(0,)),
        (1,), mode=lax.GatherScatterMode.PROMISE_IN_BOUNDS)


DTROWS = HALF // 2 + 16
CHUNK = 64
NCHUNK2 = PER_TILE_E // CHUNK


def _make_sc_layer(H, relu):
    mesh = plsc.VectorSubcoreMesh(core_axis_name="c", subcore_axis_name="s",
                                  num_cores=2, num_subcores=16)

    @functools.partial(
        pl.kernel,
        out_type=jax.ShapeDtypeStruct((NPAD, 64), F32),
        mesh=mesh,
        compiler_params=pltpu.CompilerParams(use_tc_tiling_on_sc=False),
        scratch_types=[
            pltpu.VMEM_SHARED((TROWS, 64), F32),
            pltpu.VMEM_SHARED((DTROWS, 16), F32),
            pltpu.VMEM((CHUNK,), I32),
            pltpu.VMEM((CHUNK + 16,), I32),
            pltpu.VMEM((CHUNK,), I32),
            pltpu.VMEM((CHUNK,), I32),
            pltpu.VMEM((CHUNK,), I32),
            pltpu.VMEM((CHUNK, 80), F32),
            pltpu.VMEM((CHUNK, 32), F32),
            pltpu.VMEM((CHUNK, 64), F32),
            pltpu.VMEM((CHUNK, 16), F32),
            pltpu.VMEM((CHUNK, 16), F32),
            pltpu.VMEM((16, 80), F32),
            pltpu.VMEM((16, 32), F32),
            pltpu.VMEM((16, 64), F32),
            pltpu.VMEM((8, 16), F32),
            pltpu.VMEM((16, 64), F32),
            pltpu.VMEM((64,), F32),
            pltpu.VMEM((64,), F32),
            pltpu.SemaphoreType.DMA,
            pltpu.SemaphoreType.DMA,
        ],
    )
    def sck(src_tab, dst_tab, srcv, dstv, pp, qp, hout,
            out_sp, den_sp, sidx, didx, lidx, lid2e, lid2o, srows, drows,
            msg, denb, denb2, sbuf, dbuf, abuf, dnb, obuf, pv, qv, sem1,
            sem2):
        c = lax.axis_index("c")
        s = lax.axis_index("s")
        cbase = c * HALF
        pltpu.sync_copy(pp, pv)
        pltpu.sync_copy(qp, qv)

        @pl.loop(0, CHUNK)
        def _z(r):
            zero16 = jnp.zeros((16,), F32)
            for q in range(4):
                msg[r, pl.ds(q * 16, 16)] = zero16
            denb[r, pl.ds(0, 16)] = zero16
            denb2[r, pl.ds(0, 16)] = zero16

        zb = s * 1569
        zb2 = s * 785

        @pl.loop(0, 24)
        def _zs(k):
            pltpu.sync_copy(msg, out_sp.at[pl.ds(zb + k * 64, 64)])

        pltpu.sync_copy(msg.at[pl.ds(0, 33)],
                        out_sp.at[pl.ds(zb + 1536, 33)])

        @pl.loop(0, 12)
        def _zd(k):
            pltpu.sync_copy(denb, den_sp.at[pl.ds(zb2 + k * 64, 64)])

        pltpu.sync_copy(denb.at[pl.ds(0, 17)],
                        den_sp.at[pl.ds(zb2 + 768, 17)])
        plsc.subcore_barrier()

        ebase = s * PER_TILE_E

        @pl.loop(0, NCHUNK2)
        def _chunk(i):
            b = ebase + i * CHUNK
            pltpu.sync_copy(srcv.at[pl.ds(b, CHUNK)], sidx)
            pltpu.sync_copy(dstv.at[pl.ds(b, CHUNK)],
                            didx.at[pl.ds(0, CHUNK)])
            cp1 = pltpu.async_copy(src_tab.at[sidx], srows, sem1)
            cp2 = pltpu.async_copy(dst_tab.at[didx.at[pl.ds(0, CHUNK)]],
                                   drows, sem2)
            cp1.wait()
            cp2.wait()
            for g in range(CHUNK // 16):
                dv = didx[pl.ds(g * 16, 16)]
                ld = dv - cbase
                ok = ld.astype(jnp.uint32) < jnp.uint32(HALF)
                lidx[pl.ds(g * 16, 16)] = jnp.where(ok, ld, HALF + s)
                pair = lax.shift_right_logical(ld, 1)
                oke = (ld + (ld & 1) * (2 * HALF)).astype(jnp.uint32) \
                    < jnp.uint32(HALF)
                lid2e[pl.ds(g * 16, 16)] = jnp.where(oke, pair,
                                                     HALF // 2 + s)
                oko = (ld + (1 - (ld & 1)) * (2 * HALF)) \
                    .astype(jnp.uint32) < jnp.uint32(HALF)
                lid2o[pl.ds(g * 16, 16)] = jnp.where(oko, pair,
                                                     HALF // 2 + s)

            @pl.loop(0, CHUNK)
            def _e(k):
                iota = lax.iota(I32, 16)
                iotaf = iota.astype(F32)
                lmaskf = 1.0 - jnp.minimum(
                    jnp.maximum(iotaf - (H - 1), 0.0), 1.0)
                hif = jnp.minimum(jnp.maximum(iotaf - 7.0, 0.0), 1.0)
                as_v = srows[k, pl.ds(64, 16)]
                d_v = drows[k, pl.ds(0, 16)]
                m_v = drows[k, pl.ds(8, 16)]
                al = as_v + d_v
                al = jnp.maximum(al, 0.2 * al)
                w = _sc_exp(jnp.minimum(al - m_v, 70.0)) * lmaskf
                w_hi = _lane_gather(w, jnp.maximum(iota - 8, 0)) * hif
                denb[k, pl.ds(0, 16)] = w
                denb2[k, pl.ds(0, 16)] = w_hi
                for q in range(4):
                    wq = _lane_gather(w, iota * 0 + (q if H == 4 else 0))
                    msg[k, pl.ds(q * 16, 16)] = \
                        srows[k, pl.ds(q * 16, 16)] * wq

            pltpu.sync_copy(msg, out_sp.at[lidx], add=True)
            pltpu.sync_copy(denb, den_sp.at[lid2e], add=True)
            pltpu.sync_copy(denb2, den_sp.at[lid2o], add=True)

        plsc.subcore_barrier()

        @pl.loop(0, 98)
        def _ep(k):
            l0 = s * 1568 + k * 16
            g0 = cbase + l0
            pltpu.sync_copy(src_tab.at[pl.ds(g0, 16)], sbuf)
            pltpu.sync_copy(dst_tab.at[pl.ds(g0, 16)], dbuf)
            pltpu.sync_copy(out_sp.at[pl.ds(l0, 16)], abuf)
            pltpu.sync_copy(den_sp.at[pl.ds(s * 784 + k * 8, 8)], dnb)

            @pl.loop(0, 8)
            def _n(p):
                iota = lax.iota(I32, 16)
                iotaf = iota.astype(F32)
                lmaskf = 1.0 - jnp.minimum(
                    jnp.maximum(iotaf - (H - 1), 0.0), 1.0)
                dpr = dnb[p, pl.ds(0, 16)]
                for j in range(2):
                    n = 2 * p + j
                    as_v = sbuf[n, pl.ds(64, 16)]
                    d_v = dbuf[n, pl.ds(0, 16)]
                    m_v = dbuf[n, pl.ds(8, 16)]
                    al = as_v + d_v
                    al = jnp.maximum(al, 0.2 * al)
                    ws = _sc_exp(jnp.minimum(al - m_v, 70.0)) * lmaskf
                    den = _lane_gather(dpr, (iota & 7) + 8 * j) + ws + 1e-16
                    i0 = 1.0 / den
                    inv = i0 * (2.0 - den * i0)
                    for q in range(4):
                        qh = q if H == 4 else 0
                        wq = _lane_gather(ws, iota * 0 + qh)
                        invq = _lane_gather(inv, iota * 0 + qh)
                        y = (abuf[n, pl.ds(q * 16, 16)]
                             + wq * sbuf[n, pl.ds(q * 16, 16)]) * invq
                        y = y * pv[pl.ds(q * 16, 16)] + qv[pl.ds(q * 16, 16)]
                        if relu:
                            y = jnp.maximum(y, 0.0)
                        obuf[n, pl.ds(q * 16, 16)] = y

            pltpu.sync_copy(obuf, hout.at[pl.ds(g0, 16)])

    return sck


_sc_layers = {(4, True): _make_sc_layer(4, True),
              (1, False): _make_sc_layer(1, False)}


def kernel(x, W0, b0, W1, as1, ad1, bv1, W2, as2, ad2, bv2, W3, as3, ad3,
           bv3, g1, be1, g2, be2, g3, be3, Wc1, bc1, Wc2, bc2, edge_index):
    kbn = jnp.float32(1.0) / jnp.sqrt(jnp.float32(1.0 + 1e-5))
    xpad = jnp.pad(x, ((0, NPAD - N), (0, 3)))
    w08 = jnp.pad(W0, ((0, 3), (0, 0)))
    srcv = jnp.concatenate([edge_index[0],
                            jnp.zeros((EPAD - E,), I32)])
    dstv = jnp.concatenate([edge_index[1],
                            jnp.full((EPAD - E,), NPAD - 1, I32)])

    h = _dense0(xpad, w08, b0.reshape(1, 64))

    layers = [(W1, as1, ad1, bv1, g1, be1, 4, True),
              (W2, as2, ad2, bv2, g2, be2, 4, True),
              (W3, as3, ad3, bv3, g3, be3, 1, False)]
    for (W, att_s, att_d, bv, g, be, H, relu) in layers:
        xp, as8, ad8, gmax = _pre[H](h, W, att_s.reshape(1, 64),
                                     att_d.reshape(1, 64))
        m8 = _mid(ad8, as8, gmax)
        src_tab = jnp.concatenate([xp, as8, jnp.zeros((NPAD, 8), F32)],
                                  axis=1)
        dst_tab = jnp.concatenate([ad8, m8, jnp.zeros((NPAD, 16), F32)],
                                  axis=1)
        p = kbn * g
        q = bv * kbn * g + be
        h = _sc_layers[(H, relu)](src_tab, dst_tab, srcv, dstv, p, q)

    out = _head(h, Wc1, bc1.reshape(1, 32), jnp.pad(Wc2, ((0, 0), (0, 7))),
                jnp.pad(bc2, (0, 7)).reshape(1, 8))
    return out[:N, 0]
